# baseline (speedup 1.0000x reference)
import jax
import jax.numpy as jnp
from jax import lax
from jax.experimental import pallas as pl
from jax.experimental.pallas import tpu as pltpu
from jax.experimental.pallas import tpu_sc as plsc

_NC = 2
_NS = 16
_NW = _NC * _NS

_SUB = 128
_SUBS_PER_CHUNK = 8


def _make_sc_gather(n_rows: int, d: int, rows_per_w: int, chunk: int):
    n_chunks = rows_per_w // chunk
    n_sub = chunk // _SUB

    def body(table_hbm, idx_hbm, out_hbm, idx_v, rows_v, sem):
        wid = lax.axis_index("s") * _NC + lax.axis_index("c")
        base = wid * rows_per_w

        def chunk_body(c, _):
            off = pl.multiple_of(base + c * chunk, chunk)
            pltpu.sync_copy(
                idx_hbm.at[pl.ds(pl.multiple_of(off // _SUB, n_sub), n_sub)], idx_v
            )
            cps = []
            for j in range(n_sub):
                cps.append(
                    pltpu.async_copy(
                        table_hbm.at[idx_v.at[j]],
                        rows_v.at[pl.ds(j * _SUB, _SUB)],
                        sem,
                    )
                )
            for cp in cps:
                cp.wait()
            pltpu.sync_copy(rows_v, out_hbm.at[pl.ds(off, chunk)])
            return 0

        lax.fori_loop(0, n_chunks, chunk_body, 0)

    mesh = plsc.VectorSubcoreMesh(
        core_axis_name="c", subcore_axis_name="s", num_cores=_NC, num_subcores=_NS
    )
    return pl.kernel(
        body,
        out_type=jax.ShapeDtypeStruct((n_rows, d), jnp.float32),
        mesh=mesh,
        compiler_params=pltpu.CompilerParams(use_tc_tiling_on_sc=False),
        scratch_types=[
            pltpu.VMEM((n_sub, _SUB), jnp.int32),
            pltpu.VMEM((chunk, d), jnp.float32),
            pltpu.SemaphoreType.DMA,
        ],
    )


def _tc_body(xn_ref, w_ref, nb_ref, cat_ref, cb_ref, out_ref):
    num = (
        jnp.dot(
            xn_ref[...],
            w_ref[...],
            preferred_element_type=jnp.float32,
            precision=jax.lax.Precision.HIGHEST,
        )
        + nb_ref[...]
    )
    cat = cat_ref[...] + cb_ref[...]
    out_ref[...] = jnp.concatenate([num, cat], axis=1)


def kernel(x_num, x_cat, num_w, num_b, cat_table, cat_bias):
    B, n_num, n_bins = x_num.shape
    n_cat = x_cat.shape[1]
    d_emb = cat_table.shape[1]
    card = cat_table.shape[0] // n_cat

    offsets = (jnp.arange(n_cat, dtype=jnp.int32) * card)[None]
    idx = (x_cat.astype(jnp.int32) + offsets).reshape(-1)
    n_rows = B * n_cat
    rows_per_w = n_rows // _NW
    chunk = _SUB * _SUBS_PER_CHUNK
    gather = _make_sc_gather(n_rows, d_emb, rows_per_w, chunk)
    cat_rows = gather(cat_table, idx.reshape(-1, _SUB))

    dn = n_num * d_emb
    dc = n_cat * d_emb
    eye = jnp.eye(n_num, dtype=jnp.float32)
    w_blk = (eye[:, None, :, None] * num_w[:, :, None, :]).reshape(
        n_num * n_bins, dn
    )
    bb = 512
    out2 = pl.pallas_call(
        _tc_body,
        grid=(B // bb,),
        in_specs=[
            pl.BlockSpec((bb, n_num * n_bins), lambda i: (i, 0)),
            pl.BlockSpec((n_num * n_bins, dn), lambda i: (0, 0)),
            pl.BlockSpec((1, dn), lambda i: (0, 0)),
            pl.BlockSpec((bb, dc), lambda i: (i, 0)),
            pl.BlockSpec((1, dc), lambda i: (0, 0)),
        ],
        out_specs=pl.BlockSpec((bb, dn + dc), lambda i: (i, 0)),
        out_shape=jax.ShapeDtypeStruct((B, dn + dc), jnp.float32),
    )(
        x_num.reshape(B, n_num * n_bins),
        w_blk,
        num_b.reshape(1, dn),
        cat_rows.reshape(B, dc),
        cat_bias.reshape(1, dc),
    )
    return out2.reshape(B, n_num + n_cat, d_emb)
